# baseline (speedup 1.0000x reference)
import jax
import jax.numpy as jnp
from jax import lax
from jax.experimental import pallas as pl
from jax.experimental.pallas import tpu as pltpu
from jax.experimental.pallas import tpu_sc as plsc

NC = 2
NS = 16
C = 128
SHARE0 = 0.2


def _sc_hop(table, packed, zacc, *, np_, epad):
    d = table.shape[1]
    nch_all = epad // (NS * C)
    k0 = 2 * (int(nch_all * SHARE0) // 2)
    k1 = nch_all - k0
    zr = np_ // NS
    mesh = plsc.VectorSubcoreMesh(core_axis_name="c", subcore_axis_name="s")

    def body(table_hbm, packed_hbm, zacc_hbm, out_hbm, acc_sh,
             idx0, idx1, sidx0, sidx1, rows0, rows1, gsem0, gsem1):
        c = lax.axis_index("c")
        s = lax.axis_index("s")
        idx = (idx0, idx1)
        sidx = (sidx0, sidx1)
        rows = (rows0, rows1)
        gsem = (gsem0, gsem1)

        pltpu.sync_copy(zacc_hbm.at[pl.ds(s * zr, zr)],
                        acc_sh.at[pl.ds(s * zr, zr)])
        plsc.subcore_barrier()

        nch = jnp.where(c == 0, k0, k1)
        base = (c * NS * k0 + s * nch) * C

        def stage_in(k, b):
            pltpu.sync_copy(packed_hbm.at[pl.ds((base + k * C) * 2, 2 * C)],
                            idx[b])

            def cp(i, carry):
                sidx[b][pl.ds(i * 16, 16)] = idx[b][pl.ds(C + i * 16, 16)]
                return carry

            lax.fori_loop(0, C // 16, cp, 0)
            pltpu.async_copy(table_hbm.at[idx[b].at[pl.ds(0, C)]],
                             rows[b], gsem[b])

        def drain(b):
            pltpu.make_async_copy(table_hbm.at[idx[b].at[pl.ds(0, C)]],
                                  rows[b], gsem[b]).wait()
            pltpu.sync_copy(rows[b], acc_sh.at[sidx[b]], add=True)

        stage_in(0, 0)
        P = nch // 2

        def step(p, carry):
            for b in (0, 1):
                if b == 0:
                    stage_in(2 * p + 1, 1)
                    drain(0)
                else:
                    @pl.when(p < P - 1)
                    def _():
                        stage_in(2 * p + 2, 0)
                    drain(1)
            return carry

        lax.fori_loop(0, P, step, 0)
        plsc.subcore_barrier()

        pltpu.sync_copy(acc_sh.at[pl.ds(s * zr, zr)],
                        out_hbm.at[c, pl.ds(s * zr, zr)])

    f32 = jnp.float32
    kern = pl.kernel(
        body,
        out_type=jax.ShapeDtypeStruct((NC, np_, d), f32),
        mesh=mesh,
        scratch_types=[
            pltpu.VMEM_SHARED((np_, d), f32),
            pltpu.VMEM((2 * C,), jnp.int32),
            pltpu.VMEM((2 * C,), jnp.int32),
            pltpu.VMEM((C,), jnp.int32),
            pltpu.VMEM((C,), jnp.int32),
            pltpu.VMEM((C, d), f32),
            pltpu.VMEM((C, d), f32),
            pltpu.SemaphoreType.DMA,
            pltpu.SemaphoreType.DMA,
        ],
    )
    return kern(table, packed, zacc)


def _tc_pre(bn, x, Wh):
    n, d = x.shape

    def body(x_ref, Wh_ref, out_ref):
        out_ref[...] = jnp.dot(x_ref[...], Wh_ref[...],
                               preferred_element_type=jnp.float32)

    return pl.pallas_call(
        body,
        grid=(n // bn,),
        in_specs=[
            pl.BlockSpec((bn, d), lambda i: (i, 0)),
            pl.BlockSpec((d, d), lambda i: (0, 0)),
        ],
        out_specs=pl.BlockSpec((bn, d), lambda i: (i, 0)),
        out_shape=jax.ShapeDtypeStruct((n, d), jnp.float32),
    )(x, Wh)


def _tc_combine(bn, zp, ones_col):
    _, np_, d = zp.shape
    h = d // 2

    def body(zp_ref, ones_ref, t2_ref, zh_ref):
        zsum = zp_ref[0] + zp_ref[1]
        t2_ref[...] = jnp.concatenate(
            [zsum[:, h:], ones_ref[...],
             jnp.zeros((bn, d - h - 1), jnp.float32)], axis=1)
        zh_ref[...] = zsum[:, 0:h]

    return pl.pallas_call(
        body,
        grid=(np_ // bn,),
        in_specs=[
            pl.BlockSpec((NC, bn, d), lambda i: (0, i, 0)),
            pl.BlockSpec((bn, 1), lambda i: (i, 0)),
        ],
        out_specs=[
            pl.BlockSpec((bn, d), lambda i: (i, 0)),
            pl.BlockSpec((bn, h), lambda i: (i, 0)),
        ],
        out_shape=[
            jax.ShapeDtypeStruct((np_, d), jnp.float32),
            jax.ShapeDtypeStruct((np_, h), jnp.float32),
        ],
    )(zp, ones_col)


def _tc_final(bn, x, zh, wp, esel, WeT, be, b0, b1, WcT, bc):
    n, d = x.shape
    o = WcT.shape[1]
    h = WeT.shape[1]

    def body(x_ref, zh_ref, wp_ref, esel_ref, WeT_ref, be_ref, b0_ref,
             b1_ref, WcT_ref, bc_ref, out_ref):
        he = jax.nn.relu(
            jnp.dot(x_ref[...], WeT_ref[...],
                    preferred_element_type=jnp.float32) + be_ref[...])
        wv = wp_ref[0] + wp_ref[1]
        deg = jnp.dot(wv, esel_ref[...], preferred_element_type=jnp.float32)
        dinv = 1.0 / jnp.maximum(deg, 1.0)
        h1 = jax.nn.relu(zh_ref[...] * dinv + b0_ref[...])
        h2 = jax.nn.relu(wv[:, 0:h] * (dinv * dinv) + b1_ref[...])
        wc = WcT_ref[...]
        out = (jnp.dot(he, wc[0:h], preferred_element_type=jnp.float32)
               + jnp.dot(h1, wc[h:2 * h], preferred_element_type=jnp.float32)
               + jnp.dot(h2, wc[2 * h:3 * h], preferred_element_type=jnp.float32)
               + bc_ref[...])
        out_ref[...] = out

    full = lambda shape: pl.BlockSpec(shape, lambda i: (0,) * len(shape))
    return pl.pallas_call(
        body,
        grid=(n // bn,),
        in_specs=[
            pl.BlockSpec((bn, d), lambda i: (i, 0)),
            pl.BlockSpec((bn, h), lambda i: (i, 0)),
            pl.BlockSpec((NC, bn, d), lambda i: (0, i, 0)),
            full((d, 1)),
            full((d, h)), full((1, h)),
            full((1, h)), full((1, h)),
            full((3 * h, o)), full((1, o)),
        ],
        out_specs=pl.BlockSpec((bn, o), lambda i: (i, 0)),
        out_shape=jax.ShapeDtypeStruct((n, o), jnp.float32),
    )(x, zh, wp, esel, WeT, be, b0, b1, WcT, bc)


def kernel(x, edge_index, W_ego, b_ego, W_hop0, b_hop0, W_hop1, b_hop1,
           W_cls, b_cls):
    n, d = x.shape
    e = edge_index.shape[1]
    h = W_ego.shape[0]
    np_ = -(-(n + 1) // (NS * 8)) * (NS * 8)
    per_tile = -(-e // (NC * NS * 2 * C)) * 2 * C
    epad = NC * NS * per_tile

    src = edge_index[0].astype(jnp.int32)
    dst = edge_index[1].astype(jnp.int32)
    if epad > e:
        fill = jnp.full((epad - e,), n, dtype=jnp.int32)
        src = jnp.concatenate([src, fill])
        dst = jnp.concatenate([dst, fill])
    packed = jnp.stack([dst.reshape(-1, C), src.reshape(-1, C)],
                       axis=1).reshape(-1)

    zacc = jnp.zeros((np_, d), jnp.float32)

    Wh = jnp.concatenate([W_hop0.T, W_hop1.T], axis=1)
    y1 = _tc_pre(1000, x, Wh)
    y1p = jnp.concatenate([y1, jnp.zeros((np_ - n, d), jnp.float32)], axis=0)

    zp = _sc_hop(y1p, packed, zacc, np_=np_, epad=epad)
    ones_col = jnp.ones((np_, 1), jnp.float32)
    t2, zh = _tc_combine(1264, zp, ones_col)
    wp = _sc_hop(t2, packed, zacc, np_=np_, epad=epad)

    esel = jnp.zeros((d, 1), jnp.float32).at[h, 0].set(1.0)
    return _tc_final(1000, x, zh, wp, esel,
                     W_ego.T, b_ego[None, :], b_hop0[None, :],
                     b_hop1[None, :], W_cls.T, b_cls[None, :])


# speedup vs baseline: 1.1301x; 1.1301x over previous
import jax
import jax.numpy as jnp
from jax import lax
from jax.experimental import pallas as pl
from jax.experimental.pallas import tpu as pltpu
from jax.experimental.pallas import tpu_sc as plsc

NC = 2
NS = 16
C = 128
SHARE0 = 0.77


def _sc_hop(table, packed, zacc, *, np_, epad):
    d = table.shape[1]
    nch_all = epad // (NS * C)
    k0 = 2 * (int(nch_all * SHARE0) // 2)
    k1 = nch_all - k0
    zr = np_ // NS
    mesh = plsc.VectorSubcoreMesh(core_axis_name="c", subcore_axis_name="s")

    def body(table_hbm, packed_hbm, zacc_hbm, out_hbm, acc_sh,
             idx0, idx1, sidx0, sidx1, rows0, rows1, gsem0, gsem1):
        c = lax.axis_index("c")
        s = lax.axis_index("s")
        idx = (idx0, idx1)
        sidx = (sidx0, sidx1)
        rows = (rows0, rows1)
        gsem = (gsem0, gsem1)

        pltpu.sync_copy(zacc_hbm.at[pl.ds(s * zr, zr)],
                        acc_sh.at[pl.ds(s * zr, zr)])
        plsc.subcore_barrier()

        nch = jnp.where(c == 0, k0, k1)
        base = (c * NS * k0 + s * nch) * C

        def stage_in(k, b):
            pltpu.sync_copy(packed_hbm.at[pl.ds((base + k * C) * 2, 2 * C)],
                            idx[b])

            def cp(i, carry):
                sidx[b][pl.ds(i * 16, 16)] = idx[b][pl.ds(C + i * 16, 16)]
                return carry

            lax.fori_loop(0, C // 16, cp, 0)
            pltpu.async_copy(table_hbm.at[idx[b].at[pl.ds(0, C)]],
                             rows[b], gsem[b])

        def drain(b):
            pltpu.make_async_copy(table_hbm.at[idx[b].at[pl.ds(0, C)]],
                                  rows[b], gsem[b]).wait()
            pltpu.sync_copy(rows[b], acc_sh.at[sidx[b]], add=True)

        stage_in(0, 0)
        P = nch // 2

        def step(p, carry):
            for b in (0, 1):
                if b == 0:
                    stage_in(2 * p + 1, 1)
                    drain(0)
                else:
                    @pl.when(p < P - 1)
                    def _():
                        stage_in(2 * p + 2, 0)
                    drain(1)
            return carry

        lax.fori_loop(0, P, step, 0)
        plsc.subcore_barrier()

        pltpu.sync_copy(acc_sh.at[pl.ds(s * zr, zr)],
                        out_hbm.at[c, pl.ds(s * zr, zr)])

    f32 = jnp.float32
    kern = pl.kernel(
        body,
        out_type=jax.ShapeDtypeStruct((NC, np_, d), f32),
        mesh=mesh,
        scratch_types=[
            pltpu.VMEM_SHARED((np_, d), f32),
            pltpu.VMEM((2 * C,), jnp.int32),
            pltpu.VMEM((2 * C,), jnp.int32),
            pltpu.VMEM((C,), jnp.int32),
            pltpu.VMEM((C,), jnp.int32),
            pltpu.VMEM((C, d), f32),
            pltpu.VMEM((C, d), f32),
            pltpu.SemaphoreType.DMA,
            pltpu.SemaphoreType.DMA,
        ],
    )
    return kern(table, packed, zacc)


def _tc_pre(bn, x, Wh):
    n, d = x.shape

    def body(x_ref, Wh_ref, out_ref):
        out_ref[...] = jnp.dot(x_ref[...], Wh_ref[...],
                               preferred_element_type=jnp.float32)

    return pl.pallas_call(
        body,
        grid=(n // bn,),
        in_specs=[
            pl.BlockSpec((bn, d), lambda i: (i, 0)),
            pl.BlockSpec((d, d), lambda i: (0, 0)),
        ],
        out_specs=pl.BlockSpec((bn, d), lambda i: (i, 0)),
        out_shape=jax.ShapeDtypeStruct((n, d), jnp.float32),
    )(x, Wh)


def _tc_combine(bn, zp, ones_col):
    _, np_, d = zp.shape
    h = d // 2

    def body(zp_ref, ones_ref, t2_ref, zh_ref):
        zsum = zp_ref[0] + zp_ref[1]
        t2_ref[...] = jnp.concatenate(
            [zsum[:, h:], ones_ref[...],
             jnp.zeros((bn, d - h - 1), jnp.float32)], axis=1)
        zh_ref[...] = zsum[:, 0:h]

    return pl.pallas_call(
        body,
        grid=(np_ // bn,),
        in_specs=[
            pl.BlockSpec((NC, bn, d), lambda i: (0, i, 0)),
            pl.BlockSpec((bn, 1), lambda i: (i, 0)),
        ],
        out_specs=[
            pl.BlockSpec((bn, d), lambda i: (i, 0)),
            pl.BlockSpec((bn, h), lambda i: (i, 0)),
        ],
        out_shape=[
            jax.ShapeDtypeStruct((np_, d), jnp.float32),
            jax.ShapeDtypeStruct((np_, h), jnp.float32),
        ],
    )(zp, ones_col)


def _tc_final(bn, x, zh, wp, esel, WeT, be, b0, b1, WcT, bc):
    n, d = x.shape
    o = WcT.shape[1]
    h = WeT.shape[1]

    def body(x_ref, zh_ref, wp_ref, esel_ref, WeT_ref, be_ref, b0_ref,
             b1_ref, WcT_ref, bc_ref, out_ref):
        he = jax.nn.relu(
            jnp.dot(x_ref[...], WeT_ref[...],
                    preferred_element_type=jnp.float32) + be_ref[...])
        wv = wp_ref[0] + wp_ref[1]
        deg = jnp.dot(wv, esel_ref[...], preferred_element_type=jnp.float32)
        dinv = 1.0 / jnp.maximum(deg, 1.0)
        h1 = jax.nn.relu(zh_ref[...] * dinv + b0_ref[...])
        h2 = jax.nn.relu(wv[:, 0:h] * (dinv * dinv) + b1_ref[...])
        wc = WcT_ref[...]
        out = (jnp.dot(he, wc[0:h], preferred_element_type=jnp.float32)
               + jnp.dot(h1, wc[h:2 * h], preferred_element_type=jnp.float32)
               + jnp.dot(h2, wc[2 * h:3 * h], preferred_element_type=jnp.float32)
               + bc_ref[...])
        out_ref[...] = out

    full = lambda shape: pl.BlockSpec(shape, lambda i: (0,) * len(shape))
    return pl.pallas_call(
        body,
        grid=(n // bn,),
        in_specs=[
            pl.BlockSpec((bn, d), lambda i: (i, 0)),
            pl.BlockSpec((bn, h), lambda i: (i, 0)),
            pl.BlockSpec((NC, bn, d), lambda i: (0, i, 0)),
            full((d, 1)),
            full((d, h)), full((1, h)),
            full((1, h)), full((1, h)),
            full((3 * h, o)), full((1, o)),
        ],
        out_specs=pl.BlockSpec((bn, o), lambda i: (i, 0)),
        out_shape=jax.ShapeDtypeStruct((n, o), jnp.float32),
    )(x, zh, wp, esel, WeT, be, b0, b1, WcT, bc)


def kernel(x, edge_index, W_ego, b_ego, W_hop0, b_hop0, W_hop1, b_hop1,
           W_cls, b_cls):
    n, d = x.shape
    e = edge_index.shape[1]
    h = W_ego.shape[0]
    np_ = -(-(n + 1) // (NS * 8)) * (NS * 8)
    per_tile = -(-e // (NC * NS * 2 * C)) * 2 * C
    epad = NC * NS * per_tile

    src = edge_index[0].astype(jnp.int32)
    dst = edge_index[1].astype(jnp.int32)
    if epad > e:
        fill = jnp.full((epad - e,), n, dtype=jnp.int32)
        src = jnp.concatenate([src, fill])
        dst = jnp.concatenate([dst, fill])
    packed = jnp.stack([dst.reshape(-1, C), src.reshape(-1, C)],
                       axis=1).reshape(-1)

    zacc = jnp.zeros((np_, d), jnp.float32)

    Wh = jnp.concatenate([W_hop0.T, W_hop1.T], axis=1)
    y1 = _tc_pre(1000, x, Wh)
    y1p = jnp.concatenate([y1, jnp.zeros((np_ - n, d), jnp.float32)], axis=0)

    zp = _sc_hop(y1p, packed, zacc, np_=np_, epad=epad)
    ones_col = jnp.ones((np_, 1), jnp.float32)
    t2, zh = _tc_combine(1264, zp, ones_col)
    wp = _sc_hop(t2, packed, zacc, np_=np_, epad=epad)

    esel = jnp.zeros((d, 1), jnp.float32).at[h, 0].set(1.0)
    return _tc_final(1000, x, zh, wp, esel,
                     W_ego.T, b_ego[None, :], b_hop0[None, :],
                     b_hop1[None, :], W_cls.T, b_cls[None, :])


# speedup vs baseline: 1.1459x; 1.0140x over previous
import jax
import jax.numpy as jnp
from jax import lax
from jax.experimental import pallas as pl
from jax.experimental.pallas import tpu as pltpu
from jax.experimental.pallas import tpu_sc as plsc

NC = 2
NS = 16
C = 128
SHARE0 = 0.85


def _sc_hop(table, packed, zacc, *, np_, epad):
    d = table.shape[1]
    nch_all = epad // (NS * C)
    k0 = 2 * (int(nch_all * SHARE0) // 2)
    k1 = nch_all - k0
    zr = np_ // NS
    mesh = plsc.VectorSubcoreMesh(core_axis_name="c", subcore_axis_name="s")

    def body(table_hbm, packed_hbm, zacc_hbm, out_hbm, acc_sh,
             idx0, idx1, sidx0, sidx1, rows0, rows1, gsem0, gsem1):
        c = lax.axis_index("c")
        s = lax.axis_index("s")
        idx = (idx0, idx1)
        sidx = (sidx0, sidx1)
        rows = (rows0, rows1)
        gsem = (gsem0, gsem1)

        pltpu.sync_copy(zacc_hbm.at[pl.ds(s * zr, zr)],
                        acc_sh.at[pl.ds(s * zr, zr)])
        plsc.subcore_barrier()

        nch = jnp.where(c == 0, k0, k1)
        base = (c * NS * k0 + s * nch) * C

        def stage_in(k, b):
            pltpu.sync_copy(packed_hbm.at[pl.ds((base + k * C) * 2, 2 * C)],
                            idx[b])

            def cp(i, carry):
                sidx[b][pl.ds(i * 16, 16)] = idx[b][pl.ds(C + i * 16, 16)]
                return carry

            lax.fori_loop(0, C // 16, cp, 0)
            pltpu.async_copy(table_hbm.at[idx[b].at[pl.ds(0, C)]],
                             rows[b], gsem[b])

        def drain(b):
            pltpu.make_async_copy(table_hbm.at[idx[b].at[pl.ds(0, C)]],
                                  rows[b], gsem[b]).wait()
            pltpu.sync_copy(rows[b], acc_sh.at[sidx[b]], add=True)

        stage_in(0, 0)
        P = nch // 2

        def step(p, carry):
            for b in (0, 1):
                if b == 0:
                    stage_in(2 * p + 1, 1)
                    drain(0)
                else:
                    @pl.when(p < P - 1)
                    def _():
                        stage_in(2 * p + 2, 0)
                    drain(1)
            return carry

        lax.fori_loop(0, P, step, 0)
        plsc.subcore_barrier()

        pltpu.sync_copy(acc_sh.at[pl.ds(s * zr, zr)],
                        out_hbm.at[c, pl.ds(s * zr, zr)])

    f32 = jnp.float32
    kern = pl.kernel(
        body,
        out_type=jax.ShapeDtypeStruct((NC, np_, d), f32),
        mesh=mesh,
        scratch_types=[
            pltpu.VMEM_SHARED((np_, d), f32),
            pltpu.VMEM((2 * C,), jnp.int32),
            pltpu.VMEM((2 * C,), jnp.int32),
            pltpu.VMEM((C,), jnp.int32),
            pltpu.VMEM((C,), jnp.int32),
            pltpu.VMEM((C, d), f32),
            pltpu.VMEM((C, d), f32),
            pltpu.SemaphoreType.DMA,
            pltpu.SemaphoreType.DMA,
        ],
    )
    return kern(table, packed, zacc)


def _tc_pre(bn, x, Wh):
    n, d = x.shape

    def body(x_ref, Wh_ref, out_ref):
        out_ref[...] = jnp.dot(x_ref[...], Wh_ref[...],
                               preferred_element_type=jnp.float32)

    return pl.pallas_call(
        body,
        grid=(n // bn,),
        in_specs=[
            pl.BlockSpec((bn, d), lambda i: (i, 0)),
            pl.BlockSpec((d, d), lambda i: (0, 0)),
        ],
        out_specs=pl.BlockSpec((bn, d), lambda i: (i, 0)),
        out_shape=jax.ShapeDtypeStruct((n, d), jnp.float32),
    )(x, Wh)


def _tc_combine(bn, zp, ones_col):
    _, np_, d = zp.shape
    h = d // 2

    def body(zp_ref, ones_ref, t2_ref, zh_ref):
        zsum = zp_ref[0] + zp_ref[1]
        t2_ref[...] = jnp.concatenate(
            [zsum[:, h:], ones_ref[...],
             jnp.zeros((bn, d - h - 1), jnp.float32)], axis=1)
        zh_ref[...] = zsum[:, 0:h]

    return pl.pallas_call(
        body,
        grid=(np_ // bn,),
        in_specs=[
            pl.BlockSpec((NC, bn, d), lambda i: (0, i, 0)),
            pl.BlockSpec((bn, 1), lambda i: (i, 0)),
        ],
        out_specs=[
            pl.BlockSpec((bn, d), lambda i: (i, 0)),
            pl.BlockSpec((bn, h), lambda i: (i, 0)),
        ],
        out_shape=[
            jax.ShapeDtypeStruct((np_, d), jnp.float32),
            jax.ShapeDtypeStruct((np_, h), jnp.float32),
        ],
    )(zp, ones_col)


def _tc_final(bn, x, zh, wp, esel, WeT, be, b0, b1, WcT, bc):
    n, d = x.shape
    o = WcT.shape[1]
    h = WeT.shape[1]

    def body(x_ref, zh_ref, wp_ref, esel_ref, WeT_ref, be_ref, b0_ref,
             b1_ref, WcT_ref, bc_ref, out_ref):
        he = jax.nn.relu(
            jnp.dot(x_ref[...], WeT_ref[...],
                    preferred_element_type=jnp.float32) + be_ref[...])
        wv = wp_ref[0] + wp_ref[1]
        deg = jnp.dot(wv, esel_ref[...], preferred_element_type=jnp.float32)
        dinv = 1.0 / jnp.maximum(deg, 1.0)
        h1 = jax.nn.relu(zh_ref[...] * dinv + b0_ref[...])
        h2 = jax.nn.relu(wv[:, 0:h] * (dinv * dinv) + b1_ref[...])
        wc = WcT_ref[...]
        out = (jnp.dot(he, wc[0:h], preferred_element_type=jnp.float32)
               + jnp.dot(h1, wc[h:2 * h], preferred_element_type=jnp.float32)
               + jnp.dot(h2, wc[2 * h:3 * h], preferred_element_type=jnp.float32)
               + bc_ref[...])
        out_ref[...] = out

    full = lambda shape: pl.BlockSpec(shape, lambda i: (0,) * len(shape))
    return pl.pallas_call(
        body,
        grid=(n // bn,),
        in_specs=[
            pl.BlockSpec((bn, d), lambda i: (i, 0)),
            pl.BlockSpec((bn, h), lambda i: (i, 0)),
            pl.BlockSpec((NC, bn, d), lambda i: (0, i, 0)),
            full((d, 1)),
            full((d, h)), full((1, h)),
            full((1, h)), full((1, h)),
            full((3 * h, o)), full((1, o)),
        ],
        out_specs=pl.BlockSpec((bn, o), lambda i: (i, 0)),
        out_shape=jax.ShapeDtypeStruct((n, o), jnp.float32),
    )(x, zh, wp, esel, WeT, be, b0, b1, WcT, bc)


def kernel(x, edge_index, W_ego, b_ego, W_hop0, b_hop0, W_hop1, b_hop1,
           W_cls, b_cls):
    n, d = x.shape
    e = edge_index.shape[1]
    h = W_ego.shape[0]
    np_ = -(-(n + 1) // (NS * 8)) * (NS * 8)
    per_tile = -(-e // (NC * NS * 2 * C)) * 2 * C
    epad = NC * NS * per_tile

    src = edge_index[0].astype(jnp.int32)
    dst = edge_index[1].astype(jnp.int32)
    if epad > e:
        fill = jnp.full((epad - e,), n, dtype=jnp.int32)
        src = jnp.concatenate([src, fill])
        dst = jnp.concatenate([dst, fill])
    packed = jnp.stack([dst.reshape(-1, C), src.reshape(-1, C)],
                       axis=1).reshape(-1)

    zacc = jnp.zeros((np_, d), jnp.float32)

    Wh = jnp.concatenate([W_hop0.T, W_hop1.T], axis=1)
    y1 = _tc_pre(1000, x, Wh)
    y1p = jnp.concatenate([y1, jnp.zeros((np_ - n, d), jnp.float32)], axis=0)

    zp = _sc_hop(y1p, packed, zacc, np_=np_, epad=epad)
    ones_col = jnp.ones((np_, 1), jnp.float32)
    t2, zh = _tc_combine(1264, zp, ones_col)
    wp = _sc_hop(t2, packed, zacc, np_=np_, epad=epad)

    esel = jnp.zeros((d, 1), jnp.float32).at[h, 0].set(1.0)
    return _tc_final(1000, x, zh, wp, esel,
                     W_ego.T, b_ego[None, :], b_hop0[None, :],
                     b_hop1[None, :], W_cls.T, b_cls[None, :])


# speedup vs baseline: 1.2182x; 1.0631x over previous
import jax
import jax.numpy as jnp
from jax import lax
from jax.experimental import pallas as pl
from jax.experimental.pallas import tpu as pltpu
from jax.experimental.pallas import tpu_sc as plsc

NC = 2
NS = 16
C = 128
SHARE0 = 0.95


def _sc_hop(table, packed, zacc, *, np_, epad):
    d = table.shape[1]
    nch_all = epad // (NS * C)
    k0 = 2 * (int(nch_all * SHARE0) // 2)
    k1 = nch_all - k0
    zr = np_ // NS
    mesh = plsc.VectorSubcoreMesh(core_axis_name="c", subcore_axis_name="s")

    def body(table_hbm, packed_hbm, zacc_hbm, out_hbm, acc_sh,
             idx0, idx1, sidx0, sidx1, rows0, rows1, gsem0, gsem1):
        c = lax.axis_index("c")
        s = lax.axis_index("s")
        idx = (idx0, idx1)
        sidx = (sidx0, sidx1)
        rows = (rows0, rows1)
        gsem = (gsem0, gsem1)

        pltpu.sync_copy(zacc_hbm.at[pl.ds(s * zr, zr)],
                        acc_sh.at[pl.ds(s * zr, zr)])
        plsc.subcore_barrier()

        nch = jnp.where(c == 0, k0, k1)
        base = (c * NS * k0 + s * nch) * C

        def stage_in(k, b):
            pltpu.sync_copy(packed_hbm.at[pl.ds((base + k * C) * 2, 2 * C)],
                            idx[b])

            def cp(i, carry):
                sidx[b][pl.ds(i * 16, 16)] = idx[b][pl.ds(C + i * 16, 16)]
                return carry

            lax.fori_loop(0, C // 16, cp, 0)
            pltpu.async_copy(table_hbm.at[idx[b].at[pl.ds(0, C)]],
                             rows[b], gsem[b])

        def drain(b):
            pltpu.make_async_copy(table_hbm.at[idx[b].at[pl.ds(0, C)]],
                                  rows[b], gsem[b]).wait()
            pltpu.sync_copy(rows[b], acc_sh.at[sidx[b]], add=True)

        stage_in(0, 0)
        P = nch // 2

        def step(p, carry):
            for b in (0, 1):
                if b == 0:
                    stage_in(2 * p + 1, 1)
                    drain(0)
                else:
                    @pl.when(p < P - 1)
                    def _():
                        stage_in(2 * p + 2, 0)
                    drain(1)
            return carry

        lax.fori_loop(0, P, step, 0)
        plsc.subcore_barrier()

        pltpu.sync_copy(acc_sh.at[pl.ds(s * zr, zr)],
                        out_hbm.at[c, pl.ds(s * zr, zr)])

    f32 = jnp.float32
    kern = pl.kernel(
        body,
        out_type=jax.ShapeDtypeStruct((NC, np_, d), f32),
        mesh=mesh,
        scratch_types=[
            pltpu.VMEM_SHARED((np_, d), f32),
            pltpu.VMEM((2 * C,), jnp.int32),
            pltpu.VMEM((2 * C,), jnp.int32),
            pltpu.VMEM((C,), jnp.int32),
            pltpu.VMEM((C,), jnp.int32),
            pltpu.VMEM((C, d), f32),
            pltpu.VMEM((C, d), f32),
            pltpu.SemaphoreType.DMA,
            pltpu.SemaphoreType.DMA,
        ],
    )
    return kern(table, packed, zacc)


def _tc_pre(bn, x, Wh):
    n, d = x.shape

    def body(x_ref, Wh_ref, out_ref):
        out_ref[...] = jnp.dot(x_ref[...], Wh_ref[...],
                               preferred_element_type=jnp.float32)

    return pl.pallas_call(
        body,
        grid=(n // bn,),
        in_specs=[
            pl.BlockSpec((bn, d), lambda i: (i, 0)),
            pl.BlockSpec((d, d), lambda i: (0, 0)),
        ],
        out_specs=pl.BlockSpec((bn, d), lambda i: (i, 0)),
        out_shape=jax.ShapeDtypeStruct((n, d), jnp.float32),
    )(x, Wh)


def _tc_combine(bn, zp, ones_col):
    _, np_, d = zp.shape
    h = d // 2

    def body(zp_ref, ones_ref, t2_ref, zh_ref):
        zsum = zp_ref[0] + zp_ref[1]
        t2_ref[...] = jnp.concatenate(
            [zsum[:, h:], ones_ref[...],
             jnp.zeros((bn, d - h - 1), jnp.float32)], axis=1)
        zh_ref[...] = zsum[:, 0:h]

    return pl.pallas_call(
        body,
        grid=(np_ // bn,),
        in_specs=[
            pl.BlockSpec((NC, bn, d), lambda i: (0, i, 0)),
            pl.BlockSpec((bn, 1), lambda i: (i, 0)),
        ],
        out_specs=[
            pl.BlockSpec((bn, d), lambda i: (i, 0)),
            pl.BlockSpec((bn, h), lambda i: (i, 0)),
        ],
        out_shape=[
            jax.ShapeDtypeStruct((np_, d), jnp.float32),
            jax.ShapeDtypeStruct((np_, h), jnp.float32),
        ],
    )(zp, ones_col)


def _tc_final(bn, x, zh, wp, esel, WeT, be, b0, b1, WcT, bc):
    n, d = x.shape
    o = WcT.shape[1]
    h = WeT.shape[1]

    def body(x_ref, zh_ref, wp_ref, esel_ref, WeT_ref, be_ref, b0_ref,
             b1_ref, WcT_ref, bc_ref, out_ref):
        he = jax.nn.relu(
            jnp.dot(x_ref[...], WeT_ref[...],
                    preferred_element_type=jnp.float32) + be_ref[...])
        wv = wp_ref[0] + wp_ref[1]
        deg = jnp.dot(wv, esel_ref[...], preferred_element_type=jnp.float32)
        dinv = 1.0 / jnp.maximum(deg, 1.0)
        h1 = jax.nn.relu(zh_ref[...] * dinv + b0_ref[...])
        h2 = jax.nn.relu(wv[:, 0:h] * (dinv * dinv) + b1_ref[...])
        wc = WcT_ref[...]
        out = (jnp.dot(he, wc[0:h], preferred_element_type=jnp.float32)
               + jnp.dot(h1, wc[h:2 * h], preferred_element_type=jnp.float32)
               + jnp.dot(h2, wc[2 * h:3 * h], preferred_element_type=jnp.float32)
               + bc_ref[...])
        out_ref[...] = out

    full = lambda shape: pl.BlockSpec(shape, lambda i: (0,) * len(shape))
    return pl.pallas_call(
        body,
        grid=(n // bn,),
        in_specs=[
            pl.BlockSpec((bn, d), lambda i: (i, 0)),
            pl.BlockSpec((bn, h), lambda i: (i, 0)),
            pl.BlockSpec((NC, bn, d), lambda i: (0, i, 0)),
            full((d, 1)),
            full((d, h)), full((1, h)),
            full((1, h)), full((1, h)),
            full((3 * h, o)), full((1, o)),
        ],
        out_specs=pl.BlockSpec((bn, o), lambda i: (i, 0)),
        out_shape=jax.ShapeDtypeStruct((n, o), jnp.float32),
    )(x, zh, wp, esel, WeT, be, b0, b1, WcT, bc)


def kernel(x, edge_index, W_ego, b_ego, W_hop0, b_hop0, W_hop1, b_hop1,
           W_cls, b_cls):
    n, d = x.shape
    e = edge_index.shape[1]
    h = W_ego.shape[0]
    np_ = -(-(n + 1) // (NS * 8)) * (NS * 8)
    per_tile = -(-e // (NC * NS * 2 * C)) * 2 * C
    epad = NC * NS * per_tile

    src = edge_index[0].astype(jnp.int32)
    dst = edge_index[1].astype(jnp.int32)
    if epad > e:
        fill = jnp.full((epad - e,), n, dtype=jnp.int32)
        src = jnp.concatenate([src, fill])
        dst = jnp.concatenate([dst, fill])
    packed = jnp.stack([dst.reshape(-1, C), src.reshape(-1, C)],
                       axis=1).reshape(-1)

    zacc = jnp.zeros((np_, d), jnp.float32)

    Wh = jnp.concatenate([W_hop0.T, W_hop1.T], axis=1)
    y1 = _tc_pre(1000, x, Wh)
    y1p = jnp.concatenate([y1, jnp.zeros((np_ - n, d), jnp.float32)], axis=0)

    zp = _sc_hop(y1p, packed, zacc, np_=np_, epad=epad)
    ones_col = jnp.ones((np_, 1), jnp.float32)
    t2, zh = _tc_combine(1264, zp, ones_col)
    wp = _sc_hop(t2, packed, zacc, np_=np_, epad=epad)

    esel = jnp.zeros((d, 1), jnp.float32).at[h, 0].set(1.0)
    return _tc_final(1000, x, zh, wp, esel,
                     W_ego.T, b_ego[None, :], b_hop0[None, :],
                     b_hop1[None, :], W_cls.T, b_cls[None, :])
